# baseline (speedup 1.0000x reference)
import jax
import jax.numpy as jnp
from jax import lax
from jax.experimental import pallas as pl
from jax.experimental.pallas import tpu as pltpu
from jax.experimental.pallas import tpu_sc as plsc

WIDTH = 4
LANES = 16
NW = 32
GENES_PER_W = 640
CHUNK = 128
CHUNK_GROUPS = CHUNK // LANES
N_CHUNKS = GENES_PER_W // CHUNK
ROW_UNROLL = 4
N_PARAM_ROWS = 33


def _sigm2(t):
    return 1.0 / (jnp.exp(t) + 1.0)


def _decoder_body(x_hbm, w1_hbm, b1_hbm, w2_hbm, b2_hbm, w3_hbm, b3_hbm,
                  o_hbm, xbuf, obuf, pbuf, w1r, b1r, w2r, b2r, w3r, b3r):
    n_genes = x_hbm.shape[0]
    batch = x_hbm.shape[1]
    wid = lax.axis_index("s") * 2 + lax.axis_index("c")
    g0 = jnp.minimum(wid * GENES_PER_W, n_genes - GENES_PER_W)

    iota = lax.iota(jnp.int32, LANES)
    i4 = iota * 4
    i16 = iota * 16

    def chunk_body(ci, _c):
        gc = g0 + ci * CHUNK

        pltpu.sync_copy(w1_hbm.at[pl.ds(gc * WIDTH, CHUNK * WIDTH)], w1r)
        pltpu.sync_copy(b1_hbm.at[pl.ds(gc * WIDTH, CHUNK * WIDTH)], b1r)
        pltpu.sync_copy(w2_hbm.at[pl.ds(gc * 16, CHUNK * 16)], w2r)
        pltpu.sync_copy(b2_hbm.at[pl.ds(gc * WIDTH, CHUNK * WIDTH)], b2r)
        pltpu.sync_copy(w3_hbm.at[pl.ds(gc * WIDTH, CHUNK * WIDTH)], w3r)
        pltpu.sync_copy(b3_hbm.at[pl.ds(gc, CHUNK)], b3r)
        pltpu.sync_copy(x_hbm.at[pl.ds(gc, CHUNK)],
                        xbuf.at[:, pl.ds(0, batch)])

        def reorg(gi, _):
            gs = gi * LANES
            for j in range(WIDTH):
                gw1 = plsc.load_gather(w1r, [i4 + (gs * 4 + j)])
                gb1 = plsc.load_gather(b1r, [i4 + (gs * 4 + j)])
                pbuf[j, pl.ds(gs, LANES)] = gw1 + gw1
                pbuf[4 + j, pl.ds(gs, LANES)] = gb1 + gb1
                gw2 = [plsc.load_gather(w2r, [i16 + (gs * 16 + 4 * j + k)])
                       for k in range(WIDTH)]
                for k in range(WIDTH):
                    pbuf[8 + 4 * j + k, pl.ds(gs, LANES)] = gw2[k] * (-4.0)
                gb2 = plsc.load_gather(b2r, [i4 + (gs * 4 + j)])
                w2s = (gw2[0] + gw2[1]) + (gw2[2] + gw2[3])
                pbuf[24 + j, pl.ds(gs, LANES)] = (gb2 + w2s) * 2.0
            gw3 = [plsc.load_gather(w3r, [i4 + (gs * 4 + j)])
                   for j in range(WIDTH)]
            for j in range(WIDTH):
                pbuf[28 + j, pl.ds(gs, LANES)] = gw3[j] * (-2.0)
            b3v = b3r[pl.ds(gs, LANES)]
            pbuf[32, pl.ds(gs, LANES)] = b3v + ((gw3[0] + gw3[1])
                                                + (gw3[2] + gw3[3]))
            return 0

        lax.fori_loop(0, CHUNK_GROUPS, reorg, 0)

        def group_body(gi, _1):
            gs = gi * LANES
            w1 = [pbuf[j, pl.ds(gs, LANES)] for j in range(WIDTH)]
            b1 = [pbuf[4 + j, pl.ds(gs, LANES)] for j in range(WIDTH)]
            w2 = [[pbuf[8 + 4 * j + k, pl.ds(gs, LANES)]
                   for k in range(WIDTH)] for j in range(WIDTH)]
            c2 = [pbuf[24 + j, pl.ds(gs, LANES)] for j in range(WIDTH)]
            w3 = [pbuf[28 + j, pl.ds(gs, LANES)] for j in range(WIDTH)]
            c3 = pbuf[32, pl.ds(gs, LANES)]
            rows = iota + gs

            @plsc.parallel_loop(0, batch, 1, unroll=ROW_UNROLL)
            def row_body(b):
                cols = jnp.full((LANES,), b, jnp.int32)
                x = plsc.load_gather(xbuf, [rows, cols])
                u = [_sigm2(x * w1[j] + b1[j]) for j in range(WIDTH)]
                v = []
                for j in range(WIDTH):
                    s = c2[j]
                    for k in range(WIDTH):
                        s = s + u[k] * w2[j][k]
                    v.append(_sigm2(s))
                o = c3
                for j in range(WIDTH):
                    o = o + v[j] * w3[j]
                plsc.store_scatter(obuf, [rows, cols], o)

            return 0

        lax.fori_loop(0, CHUNK_GROUPS, group_body, 0)
        pltpu.sync_copy(obuf.at[:, pl.ds(0, batch)], o_hbm.at[pl.ds(gc, CHUNK)])
        return 0

    lax.fori_loop(0, N_CHUNKS, chunk_body, 0)


def _build(n_genes, batch, interpret=False):
    mesh = plsc.VectorSubcoreMesh(core_axis_name="c", subcore_axis_name="s")
    return pl.kernel(
        _decoder_body,
        out_type=jax.ShapeDtypeStruct((n_genes, batch), jnp.float32),
        mesh=mesh,
        scratch_types=[
            pltpu.VMEM((CHUNK, batch + 8), jnp.float32),
            pltpu.VMEM((CHUNK, batch + 8), jnp.float32),
            pltpu.VMEM((N_PARAM_ROWS, CHUNK), jnp.float32),
            pltpu.VMEM((CHUNK * WIDTH,), jnp.float32),
            pltpu.VMEM((CHUNK * WIDTH,), jnp.float32),
            pltpu.VMEM((CHUNK * 16,), jnp.float32),
            pltpu.VMEM((CHUNK * WIDTH,), jnp.float32),
            pltpu.VMEM((CHUNK * WIDTH,), jnp.float32),
            pltpu.VMEM((CHUNK,), jnp.float32),
        ],
        compiler_params=pltpu.CompilerParams(needs_layout_passes=False),
        interpret=interpret,
    )


def kernel(features, w1, b1, w2, b2, w3, b3, r1, c1, r2, c2, r3, c3):
    batch, n_genes = features.shape
    f = _build(n_genes, batch)
    return f(features.T, w1, b1, w2, b2, w3, b3).T


# speedup vs baseline: 1.4586x; 1.4586x over previous
import jax
import jax.numpy as jnp
from jax import lax
from jax.experimental import pallas as pl
from jax.experimental.pallas import tpu as pltpu
from jax.experimental.pallas import tpu_sc as plsc

WIDTH = 4
LANES = 16
NW = 32
GENES_PER_W = 640
GROUPS_PER_W = GENES_PER_W // LANES
ROW_BLOCK = 32
ROW_UNROLL = 4
N_PARAM_ROWS = 33


def _sigm2(t):
    return 1.0 / (jnp.exp(t) + 1.0)


def _stage_params(g0, n, w1_hbm, b1_hbm, w2_hbm, b2_hbm, w3_hbm, b3_hbm,
                  pbuf, w1r, b1r, w2r, b2r, w3r, b3r):
    pltpu.sync_copy(w1_hbm.at[pl.ds(g0 * WIDTH, n * WIDTH)],
                    w1r.at[pl.ds(0, n * WIDTH)])
    pltpu.sync_copy(b1_hbm.at[pl.ds(g0 * WIDTH, n * WIDTH)],
                    b1r.at[pl.ds(0, n * WIDTH)])
    pltpu.sync_copy(w2_hbm.at[pl.ds(g0 * 16, n * 16)],
                    w2r.at[pl.ds(0, n * 16)])
    pltpu.sync_copy(b2_hbm.at[pl.ds(g0 * WIDTH, n * WIDTH)],
                    b2r.at[pl.ds(0, n * WIDTH)])
    pltpu.sync_copy(w3_hbm.at[pl.ds(g0 * WIDTH, n * WIDTH)],
                    w3r.at[pl.ds(0, n * WIDTH)])
    pltpu.sync_copy(b3_hbm.at[pl.ds(g0, n)], b3r.at[pl.ds(0, n)])


def _make_reorg(pbuf, w1r, b1r, w2r, b2r, w3r, b3r):
    iota = lax.iota(jnp.int32, LANES)
    i4 = iota * 4
    i16 = iota * 16

    def reorg(gi, _):
        gs = gi * LANES
        for j in range(WIDTH):
            gw1 = plsc.load_gather(w1r, [i4 + (gs * 4 + j)])
            gb1 = plsc.load_gather(b1r, [i4 + (gs * 4 + j)])
            pbuf[j, pl.ds(gs, LANES)] = gw1 + gw1
            pbuf[4 + j, pl.ds(gs, LANES)] = gb1 + gb1
            gw2 = [plsc.load_gather(w2r, [i16 + (gs * 16 + 4 * j + k)])
                   for k in range(WIDTH)]
            for k in range(WIDTH):
                pbuf[8 + 4 * j + k, pl.ds(gs, LANES)] = gw2[k] * (-4.0)
            gb2 = plsc.load_gather(b2r, [i4 + (gs * 4 + j)])
            w2s = (gw2[0] + gw2[1]) + (gw2[2] + gw2[3])
            pbuf[24 + j, pl.ds(gs, LANES)] = (gb2 + w2s) * 2.0
        gw3 = [plsc.load_gather(w3r, [i4 + (gs * 4 + j)])
               for j in range(WIDTH)]
        for j in range(WIDTH):
            pbuf[28 + j, pl.ds(gs, LANES)] = gw3[j] * (-2.0)
        b3v = b3r[pl.ds(gs, LANES)]
        pbuf[32, pl.ds(gs, LANES)] = b3v + ((gw3[0] + gw3[1])
                                            + (gw3[2] + gw3[3]))
        return 0

    return reorg


def _mlp_block(xref, oref, pbuf, n_groups, n_rows):
    def group_body(gi, _1):
        gs = gi * LANES
        w1 = [pbuf[j, pl.ds(gs, LANES)] for j in range(WIDTH)]
        b1 = [pbuf[4 + j, pl.ds(gs, LANES)] for j in range(WIDTH)]
        w2 = [[pbuf[8 + 4 * j + k, pl.ds(gs, LANES)] for k in range(WIDTH)]
              for j in range(WIDTH)]
        c2 = [pbuf[24 + j, pl.ds(gs, LANES)] for j in range(WIDTH)]
        w3 = [pbuf[28 + j, pl.ds(gs, LANES)] for j in range(WIDTH)]
        c3 = pbuf[32, pl.ds(gs, LANES)]

        @plsc.parallel_loop(0, n_rows, 1, unroll=ROW_UNROLL)
        def row_body(b):
            x = xref[b, pl.ds(gs, LANES)]
            u = [_sigm2(x * w1[j] + b1[j]) for j in range(WIDTH)]
            v = []
            for j in range(WIDTH):
                s = c2[j]
                for k in range(WIDTH):
                    s = s + u[k] * w2[j][k]
                v.append(_sigm2(s))
            o = c3
            for j in range(WIDTH):
                o = o + v[j] * w3[j]
            oref[b, pl.ds(gs, LANES)] = o

        return 0

    lax.fori_loop(0, n_groups, group_body, 0)


def _decoder_body(x_hbm, w1_hbm, b1_hbm, w2_hbm, b2_hbm, w3_hbm, b3_hbm,
                  o_hbm, xbuf, obuf, pbuf, w1r, b1r, w2r, b2r, w3r, b3r,
                  xtail, otail):
    batch = x_hbm.shape[0]
    n_genes = x_hbm.shape[1]
    wid = lax.axis_index("s") * 2 + lax.axis_index("c")
    g_last = ((n_genes - GENES_PER_W) // 128) * 128
    g0 = jnp.minimum(wid * GENES_PER_W, g_last)

    params = (w1_hbm, b1_hbm, w2_hbm, b2_hbm, w3_hbm, b3_hbm)
    bufs = (pbuf, w1r, b1r, w2r, b2r, w3r, b3r)
    _stage_params(g0, GENES_PER_W, *params, *bufs)
    lax.fori_loop(0, GROUPS_PER_W, _make_reorg(*bufs), 0)

    def block_body(rb, _0):
        r0 = rb * ROW_BLOCK
        pltpu.sync_copy(x_hbm.at[pl.ds(r0, ROW_BLOCK), pl.ds(g0, GENES_PER_W)],
                        xbuf)
        _mlp_block(xbuf, obuf, pbuf, GROUPS_PER_W, ROW_BLOCK)
        pltpu.sync_copy(obuf,
                        o_hbm.at[pl.ds(r0, ROW_BLOCK), pl.ds(g0, GENES_PER_W)])
        return 0

    lax.fori_loop(0, batch // ROW_BLOCK, block_body, 0)

    tail0 = g_last + GENES_PER_W
    tail_len = n_genes - tail0
    if tail_len > 0:
        tail_rows = batch // NW
        tail_groups = tail_len // LANES
        _stage_params(tail0, tail_len, *params, *bufs)
        lax.fori_loop(0, tail_groups, _make_reorg(*bufs), 0)
        rt = wid * tail_rows
        pltpu.sync_copy(x_hbm.at[pl.ds(rt, tail_rows), pl.ds(tail0, tail_len)],
                        xtail)
        _mlp_block(xtail, otail, pbuf, tail_groups, tail_rows)
        pltpu.sync_copy(otail,
                        o_hbm.at[pl.ds(rt, tail_rows), pl.ds(tail0, tail_len)])


def _build(batch, n_genes, interpret=False):
    mesh = plsc.VectorSubcoreMesh(core_axis_name="c", subcore_axis_name="s")
    g_last = ((n_genes - GENES_PER_W) // 128) * 128
    tail_len = n_genes - (g_last + GENES_PER_W)
    tail_rows = max(batch // NW, 1)
    return pl.kernel(
        _decoder_body,
        out_type=jax.ShapeDtypeStruct((batch, n_genes), jnp.float32),
        mesh=mesh,
        scratch_types=[
            pltpu.VMEM((ROW_BLOCK, GENES_PER_W), jnp.float32),
            pltpu.VMEM((ROW_BLOCK, GENES_PER_W), jnp.float32),
            pltpu.VMEM((N_PARAM_ROWS, GENES_PER_W), jnp.float32),
            pltpu.VMEM((GENES_PER_W * WIDTH,), jnp.float32),
            pltpu.VMEM((GENES_PER_W * WIDTH,), jnp.float32),
            pltpu.VMEM((GENES_PER_W * 16,), jnp.float32),
            pltpu.VMEM((GENES_PER_W * WIDTH,), jnp.float32),
            pltpu.VMEM((GENES_PER_W * WIDTH,), jnp.float32),
            pltpu.VMEM((GENES_PER_W,), jnp.float32),
            pltpu.VMEM((tail_rows, max(tail_len, LANES)), jnp.float32),
            pltpu.VMEM((tail_rows, max(tail_len, LANES)), jnp.float32),
        ],
        compiler_params=pltpu.CompilerParams(needs_layout_passes=False),
        interpret=interpret,
    )


def kernel(features, w1, b1, w2, b2, w3, b3, r1, c1, r2, c2, r3, c3):
    batch, n_genes = features.shape
    f = _build(batch, n_genes)
    return f(features, w1, b1, w2, b2, w3, b3)


# speedup vs baseline: 1.5057x; 1.0323x over previous
import jax
import jax.numpy as jnp
from jax import lax
from jax.experimental import pallas as pl
from jax.experimental.pallas import tpu as pltpu
from jax.experimental.pallas import tpu_sc as plsc

WIDTH = 4
LANES = 16
NW = 32
GENES_PER_W = 640
GROUPS_PER_W = GENES_PER_W // LANES
ROW_BLOCK = 32
ROW_UNROLL = 4
N_PARAM_ROWS = 33


def _sigm2(t):
    return 1.0 / (jnp.exp(t) + 1.0)


def _stage_params(g0, n, w1_hbm, b1_hbm, w2_hbm, b2_hbm, w3_hbm, b3_hbm,
                  pbuf, w1r, b1r, w2r, b2r, w3r, b3r):
    pltpu.sync_copy(w1_hbm.at[pl.ds(g0 * WIDTH, n * WIDTH)],
                    w1r.at[pl.ds(0, n * WIDTH)])
    pltpu.sync_copy(b1_hbm.at[pl.ds(g0 * WIDTH, n * WIDTH)],
                    b1r.at[pl.ds(0, n * WIDTH)])
    pltpu.sync_copy(w2_hbm.at[pl.ds(g0 * 16, n * 16)],
                    w2r.at[pl.ds(0, n * 16)])
    pltpu.sync_copy(b2_hbm.at[pl.ds(g0 * WIDTH, n * WIDTH)],
                    b2r.at[pl.ds(0, n * WIDTH)])
    pltpu.sync_copy(w3_hbm.at[pl.ds(g0 * WIDTH, n * WIDTH)],
                    w3r.at[pl.ds(0, n * WIDTH)])
    pltpu.sync_copy(b3_hbm.at[pl.ds(g0, n)], b3r.at[pl.ds(0, n)])


def _make_reorg(pbuf, w1r, b1r, w2r, b2r, w3r, b3r):
    iota = lax.iota(jnp.int32, LANES)
    i4 = iota * 4
    i16 = iota * 16

    def reorg(gi, _):
        gs = gi * LANES
        for j in range(WIDTH):
            gw1 = plsc.load_gather(w1r, [i4 + (gs * 4 + j)])
            gb1 = plsc.load_gather(b1r, [i4 + (gs * 4 + j)])
            pbuf[j, pl.ds(gs, LANES)] = gw1 + gw1
            pbuf[4 + j, pl.ds(gs, LANES)] = gb1 + gb1
            gw2 = [plsc.load_gather(w2r, [i16 + (gs * 16 + 4 * j + k)])
                   for k in range(WIDTH)]
            for k in range(WIDTH):
                pbuf[8 + 4 * j + k, pl.ds(gs, LANES)] = gw2[k] * (-4.0)
            gb2 = plsc.load_gather(b2r, [i4 + (gs * 4 + j)])
            w2s = (gw2[0] + gw2[1]) + (gw2[2] + gw2[3])
            pbuf[24 + j, pl.ds(gs, LANES)] = (gb2 + w2s) * 2.0
        gw3 = [plsc.load_gather(w3r, [i4 + (gs * 4 + j)])
               for j in range(WIDTH)]
        for j in range(WIDTH):
            pbuf[28 + j, pl.ds(gs, LANES)] = gw3[j] * (-2.0)
        b3v = b3r[pl.ds(gs, LANES)]
        pbuf[32, pl.ds(gs, LANES)] = b3v + ((gw3[0] + gw3[1])
                                            + (gw3[2] + gw3[3]))
        return 0

    return reorg


def _mlp_block(xref, oref, pbuf, n_groups, n_rows):
    def group_body(gi, _1):
        gs = gi * LANES
        w1 = [pbuf[j, pl.ds(gs, LANES)] for j in range(WIDTH)]
        b1 = [pbuf[4 + j, pl.ds(gs, LANES)] for j in range(WIDTH)]
        w2 = [[pbuf[8 + 4 * j + k, pl.ds(gs, LANES)] for k in range(WIDTH)]
              for j in range(WIDTH)]
        c2 = [pbuf[24 + j, pl.ds(gs, LANES)] for j in range(WIDTH)]
        w3 = [pbuf[28 + j, pl.ds(gs, LANES)] for j in range(WIDTH)]
        c3 = pbuf[32, pl.ds(gs, LANES)]

        @plsc.parallel_loop(0, n_rows, 1, unroll=ROW_UNROLL)
        def row_body(b):
            x = xref[b, pl.ds(gs, LANES)]
            u = [_sigm2(x * w1[j] + b1[j]) for j in range(WIDTH)]
            v = []
            for j in range(WIDTH):
                s = c2[j]
                for k in range(WIDTH):
                    s = s + u[k] * w2[j][k]
                v.append(_sigm2(s))
            o = c3
            for j in range(WIDTH):
                o = o + v[j] * w3[j]
            oref[b, pl.ds(gs, LANES)] = o

        return 0

    lax.fori_loop(0, n_groups, group_body, 0)


def _decoder_body(x_hbm, w1_hbm, b1_hbm, w2_hbm, b2_hbm, w3_hbm, b3_hbm,
                  o_hbm, xbuf, xbuf2, obuf, pbuf,
                  w1r, b1r, w2r, b2r, w3r, b3r, xtail, otail, xsem):
    batch = x_hbm.shape[0]
    n_genes = x_hbm.shape[1]
    wid = lax.axis_index("s") * 2 + lax.axis_index("c")
    g_last = ((n_genes - GENES_PER_W) // 128) * 128
    g0 = jnp.minimum(wid * GENES_PER_W, g_last)

    params = (w1_hbm, b1_hbm, w2_hbm, b2_hbm, w3_hbm, b3_hbm)
    bufs = (pbuf, w1r, b1r, w2r, b2r, w3r, b3r)
    _stage_params(g0, GENES_PER_W, *params, *bufs)
    lax.fori_loop(0, GROUPS_PER_W, _make_reorg(*bufs), 0)

    n_blocks = batch // ROW_BLOCK
    xv = (xbuf, xbuf2)

    def xsl(rb):
        return x_hbm.at[pl.ds(rb * ROW_BLOCK, ROW_BLOCK),
                        pl.ds(g0, GENES_PER_W)]

    def osl(rb):
        return o_hbm.at[pl.ds(rb * ROW_BLOCK, ROW_BLOCK),
                        pl.ds(g0, GENES_PER_W)]

    cp0 = pltpu.async_copy(xsl(0), xv[0], xsem)
    for rb in range(n_blocks):
        cur = rb % 2
        if rb + 1 < n_blocks:
            nxt = pltpu.async_copy(xsl(rb + 1), xv[1 - cur], xsem)
        cp0.wait()
        _mlp_block(xv[cur], obuf, pbuf, GROUPS_PER_W, ROW_BLOCK)
        pltpu.sync_copy(obuf, osl(rb))
        if rb + 1 < n_blocks:
            cp0 = nxt

    tail0 = g_last + GENES_PER_W
    tail_len = n_genes - tail0
    if tail_len > 0:
        tail_rows = batch // NW
        tail_groups = tail_len // LANES
        _stage_params(tail0, tail_len, *params, *bufs)
        lax.fori_loop(0, tail_groups, _make_reorg(*bufs), 0)
        rt = wid * tail_rows
        pltpu.sync_copy(x_hbm.at[pl.ds(rt, tail_rows), pl.ds(tail0, tail_len)],
                        xtail)
        _mlp_block(xtail, otail, pbuf, tail_groups, tail_rows)
        pltpu.sync_copy(otail,
                        o_hbm.at[pl.ds(rt, tail_rows), pl.ds(tail0, tail_len)])


def _build(batch, n_genes, interpret=False):
    mesh = plsc.VectorSubcoreMesh(core_axis_name="c", subcore_axis_name="s")
    g_last = ((n_genes - GENES_PER_W) // 128) * 128
    tail_len = n_genes - (g_last + GENES_PER_W)
    tail_rows = max(batch // NW, 1)
    return pl.kernel(
        _decoder_body,
        out_type=jax.ShapeDtypeStruct((batch, n_genes), jnp.float32),
        mesh=mesh,
        scratch_types=[
            pltpu.VMEM((ROW_BLOCK, GENES_PER_W), jnp.float32),
            pltpu.VMEM((ROW_BLOCK, GENES_PER_W), jnp.float32),
            pltpu.VMEM((ROW_BLOCK, GENES_PER_W), jnp.float32),
            pltpu.VMEM((N_PARAM_ROWS, GENES_PER_W), jnp.float32),
            pltpu.VMEM((GENES_PER_W * WIDTH,), jnp.float32),
            pltpu.VMEM((GENES_PER_W * WIDTH,), jnp.float32),
            pltpu.VMEM((GENES_PER_W * 16,), jnp.float32),
            pltpu.VMEM((GENES_PER_W * WIDTH,), jnp.float32),
            pltpu.VMEM((GENES_PER_W * WIDTH,), jnp.float32),
            pltpu.VMEM((GENES_PER_W,), jnp.float32),
            pltpu.VMEM((tail_rows, max(tail_len, LANES)), jnp.float32),
            pltpu.VMEM((tail_rows, max(tail_len, LANES)), jnp.float32),
            pltpu.SemaphoreType.DMA,
        ],
        compiler_params=pltpu.CompilerParams(needs_layout_passes=False),
        interpret=interpret,
    )


def kernel(features, w1, b1, w2, b2, w3, b3, r1, c1, r2, c2, r3, c3):
    batch, n_genes = features.shape
    f = _build(batch, n_genes)
    return f(features, w1, b1, w2, b2, w3, b3)
